# baseline (speedup 1.0000x reference)
import functools

import jax
import jax.numpy as jnp
from jax import lax
from jax.experimental import pallas as pl
from jax.experimental.pallas import tpu as pltpu
from jax.experimental.pallas import tpu_sc as plsc

_INFO = plsc.get_sparse_core_info()
_NC = _INFO.num_cores
_NS = _INFO.num_subcores
_NW = _NC * _NS
_CHUNK = 64
_GC = 4
_GROUP = _CHUNK * _GC
_NSLOT = 3


def _embed_lookup(idx_grouped, W, n_rows, d):
    k = idx_grouped.shape[1]
    per_w = k * _CHUNK
    ng = k // _GC
    assert ng > _NSLOT + 2

    @functools.partial(
        pl.kernel,
        mesh=plsc.VectorSubcoreMesh(core_axis_name="c", subcore_axis_name="s"),
        out_type=jax.ShapeDtypeStruct((n_rows, d), jnp.float32),
        scratch_types=[
            pltpu.VMEM((k, _CHUNK), jnp.int32),
            pltpu.VMEM((_NSLOT, _GROUP, d), jnp.float32),
            pltpu.SemaphoreType.DMA((_NSLOT,)),
            pltpu.SemaphoreType.DMA((_NSLOT,)),
        ],
    )
    def emb(idx_hbm, table_hbm, out_hbm, idx_v, bufs, gsem, wsem):
        wid = lax.axis_index("s") * _NC + lax.axis_index("c")
        base = wid * per_w
        pltpu.sync_copy(idx_hbm.at[wid], idx_v)

        def start_group_gathers(g, s):
            for i in range(_GC):
                pltpu.async_copy(
                    table_hbm.at[idx_v.at[g * _GC + i]],
                    bufs.at[s, pl.ds(i * _CHUNK, _CHUNK)], gsem.at[s])

        def wait_group_gathers(s):
            pltpu.make_async_copy(
                table_hbm.at[pl.ds(0, _GROUP)], bufs.at[s], gsem.at[s]).wait()

        def start_write(g, s):
            pltpu.async_copy(
                bufs.at[s], out_hbm.at[pl.ds(base + g * _GROUP, _GROUP)],
                wsem.at[s])

        def wait_write(s):
            pltpu.make_async_copy(
                bufs.at[s], out_hbm.at[pl.ds(0, _GROUP)], wsem.at[s]).wait()

        start_group_gathers(0, 0)
        start_group_gathers(1, 1)

        wait_group_gathers(0)
        start_write(0, 0)
        start_group_gathers(2, 2)

        wait_group_gathers(1)
        start_write(1, 1)
        wait_write(0)
        start_group_gathers(3, 0)

        nt = (ng - 2) // _NSLOT

        def body(t, carry):
            h0 = 2 + _NSLOT * t
            for u in range(_NSLOT):
                h = h0 + u
                s = (2 + u) % _NSLOT
                r = (4 + u) % _NSLOT
                wait_group_gathers(s)
                start_write(h, s)

                @pl.when(h < ng - 2)
                def _():
                    wait_write(r)
                    start_group_gathers(h + 2, r)
            return carry

        lax.fori_loop(0, nt, body, 0)
        for h in range(2 + _NSLOT * nt, ng):
            wait_group_gathers(h % _NSLOT)
            start_write(h, h % _NSLOT)
        for g in range(ng - _NSLOT, ng):
            wait_write(g % _NSLOT)

    return emb(idx_grouped, W)


def kernel(input_ids, W):
    B, L = input_ids.shape
    V, D = W.shape
    n = B * L
    idx = input_ids.reshape(_NW, n // (_NW * _CHUNK), _CHUNK).astype(jnp.int32)
    out = _embed_lookup(idx, W, n, D)
    return out.reshape(B, L, D)


# speedup vs baseline: 1.0202x; 1.0202x over previous
import functools

import jax
import jax.numpy as jnp
from jax import lax
from jax.experimental import pallas as pl
from jax.experimental.pallas import tpu as pltpu
from jax.experimental.pallas import tpu_sc as plsc

_INFO = plsc.get_sparse_core_info()
_NC = _INFO.num_cores
_NS = _INFO.num_subcores
_NW = _NC * _NS
_CHUNK = 80
_NBUF = 8
_AHEAD = 5


def _embed_lookup(idx_grouped, W, n_rows, d):
    k = idx_grouped.shape[1]
    per_w = k * _CHUNK
    assert k % _NBUF == 0
    nm = k // _NBUF

    @functools.partial(
        pl.kernel,
        mesh=plsc.VectorSubcoreMesh(core_axis_name="c", subcore_axis_name="s"),
        out_type=jax.ShapeDtypeStruct((n_rows, d), jnp.float32),
        scratch_types=[
            pltpu.VMEM((k, _CHUNK), jnp.int32),
            pltpu.VMEM((_NBUF, _CHUNK, d), jnp.float32),
            pltpu.SemaphoreType.DMA((_NBUF,)),
            pltpu.SemaphoreType.DMA((_NBUF,)),
        ],
    )
    def emb(idx_hbm, table_hbm, out_hbm, idx_v, bufs, gsem, wsem):
        wid = lax.axis_index("s") * _NC + lax.axis_index("c")
        base = wid * per_w
        pltpu.sync_copy(idx_hbm.at[wid], idx_v)

        def start_gather(j, b):
            pltpu.async_copy(table_hbm.at[idx_v.at[j]], bufs.at[b], gsem.at[b])

        def wait_gather(b):
            pltpu.make_async_copy(
                table_hbm.at[pl.ds(0, _CHUNK)], bufs.at[b], gsem.at[b]).wait()

        def start_write(j, b):
            pltpu.async_copy(
                bufs.at[b], out_hbm.at[pl.ds(base + j * _CHUNK, _CHUNK)],
                wsem.at[b])

        def wait_write(b):
            pltpu.make_async_copy(
                bufs.at[b], out_hbm.at[pl.ds(0, _CHUNK)], wsem.at[b]).wait()

        for b in range(_AHEAD):
            start_gather(b, b)

        def body(m, carry):
            j0 = _NBUF * m
            for u in range(_NBUF):
                j = j0 + u
                pb = (u + _AHEAD) % _NBUF
                wait_gather(u)
                if u < _NBUF - _AHEAD:
                    @pl.when(m > 0)
                    def _():
                        wait_write(pb)
                        start_gather(j + _AHEAD, pb)

                    @pl.when(m == 0)
                    def _():
                        start_gather(j + _AHEAD, pb)
                else:
                    wait_write(pb)

                    @pl.when(m < nm - 1)
                    def _():
                        start_gather(j + _AHEAD, pb)
                start_write(j, u)
            return carry

        lax.fori_loop(0, nm, body, 0)
        for j in range(k - (_NBUF - _AHEAD), k):
            wait_write(j % _NBUF)

    return emb(idx_grouped, W)


def kernel(input_ids, W):
    B, L = input_ids.shape
    V, D = W.shape
    n = B * L
    idx = input_ids.reshape(_NW, n // (_NW * _CHUNK), _CHUNK).astype(jnp.int32)
    out = _embed_lookup(idx, W, n, D)
    return out.reshape(B, L, D)


# speedup vs baseline: 1.0206x; 1.0004x over previous
import functools

import jax
import jax.numpy as jnp
from jax import lax
from jax.experimental import pallas as pl
from jax.experimental.pallas import tpu as pltpu
from jax.experimental.pallas import tpu_sc as plsc

_INFO = plsc.get_sparse_core_info()
_NC = _INFO.num_cores
_NS = _INFO.num_subcores
_NW = _NC * _NS
_CHUNK = 80
_NBUF = 8
_AHEAD = 4


def _embed_lookup(idx_grouped, W, n_rows, d):
    k = idx_grouped.shape[1]
    per_w = k * _CHUNK
    assert k % _NBUF == 0
    nm = k // _NBUF

    @functools.partial(
        pl.kernel,
        mesh=plsc.VectorSubcoreMesh(core_axis_name="c", subcore_axis_name="s"),
        out_type=jax.ShapeDtypeStruct((n_rows, d), jnp.float32),
        scratch_types=[
            pltpu.VMEM((k, _CHUNK), jnp.int32),
            pltpu.VMEM((_NBUF, _CHUNK, d), jnp.float32),
            pltpu.SemaphoreType.DMA((_NBUF,)),
            pltpu.SemaphoreType.DMA((_NBUF,)),
        ],
    )
    def emb(idx_hbm, table_hbm, out_hbm, idx_v, bufs, gsem, wsem):
        wid = lax.axis_index("s") * _NC + lax.axis_index("c")
        base = wid * per_w
        pltpu.sync_copy(idx_hbm.at[wid], idx_v)

        def start_gather(j, b):
            pltpu.async_copy(table_hbm.at[idx_v.at[j]], bufs.at[b], gsem.at[b])

        def wait_gather(b):
            pltpu.make_async_copy(
                table_hbm.at[pl.ds(0, _CHUNK)], bufs.at[b], gsem.at[b]).wait()

        def start_write(j, b):
            pltpu.async_copy(
                bufs.at[b], out_hbm.at[pl.ds(base + j * _CHUNK, _CHUNK)],
                wsem.at[b])

        def wait_write(b):
            pltpu.make_async_copy(
                bufs.at[b], out_hbm.at[pl.ds(0, _CHUNK)], wsem.at[b]).wait()

        for b in range(_AHEAD):
            start_gather(b, b)

        def body(m, carry):
            j0 = _NBUF * m
            for u in range(_NBUF):
                j = j0 + u
                pb = (u + _AHEAD) % _NBUF
                wait_gather(u)
                if u < _NBUF - _AHEAD:
                    @pl.when(m > 0)
                    def _():
                        wait_write(pb)
                        start_gather(j + _AHEAD, pb)

                    @pl.when(m == 0)
                    def _():
                        start_gather(j + _AHEAD, pb)
                else:
                    wait_write(pb)

                    @pl.when(m < nm - 1)
                    def _():
                        start_gather(j + _AHEAD, pb)
                start_write(j, u)
            return carry

        lax.fori_loop(0, nm, body, 0)
        for j in range(k - (_NBUF - _AHEAD), k):
            wait_write(j % _NBUF)

    return emb(idx_grouped, W)


def kernel(input_ids, W):
    B, L = input_ids.shape
    V, D = W.shape
    n = B * L
    idx = input_ids.reshape(_NW, n // (_NW * _CHUNK), _CHUNK).astype(jnp.int32)
    out = _embed_lookup(idx, W, n, D)
    return out.reshape(B, L, D)


# speedup vs baseline: 1.0210x; 1.0004x over previous
import functools

import jax
import jax.numpy as jnp
from jax import lax
from jax.experimental import pallas as pl
from jax.experimental.pallas import tpu as pltpu
from jax.experimental.pallas import tpu_sc as plsc

_INFO = plsc.get_sparse_core_info()
_NC = _INFO.num_cores
_NS = _INFO.num_subcores
_NW = _NC * _NS
_CHUNK = 80
_NBUF = 8
_AHEAD = 5


def _embed_lookup(idx_grouped, W, n_rows, d):
    k = idx_grouped.shape[1]
    per_w = k * _CHUNK
    assert k % _NBUF == 0
    nm = k // _NBUF

    @functools.partial(
        pl.kernel,
        mesh=plsc.VectorSubcoreMesh(core_axis_name="c", subcore_axis_name="s"),
        out_type=jax.ShapeDtypeStruct((n_rows, d), jnp.float32),
        scratch_types=[
            pltpu.VMEM((k, _CHUNK), jnp.int32),
            pltpu.VMEM((_NBUF, _CHUNK, d), jnp.float32),
            pltpu.SemaphoreType.DMA((_NBUF,)),
            pltpu.SemaphoreType.DMA((_NBUF,)),
        ],
    )
    def emb(idx_hbm, table_hbm, out_hbm, idx_v, bufs, gsem, wsem):
        wid = lax.axis_index("s") * _NC + lax.axis_index("c")
        base = wid * per_w
        pltpu.sync_copy(idx_hbm.at[wid], idx_v)

        def start_gather(j, b):
            pltpu.async_copy(table_hbm.at[idx_v.at[j]], bufs.at[b], gsem.at[b])

        def wait_gather(b):
            pltpu.make_async_copy(
                table_hbm.at[pl.ds(0, _CHUNK)], bufs.at[b], gsem.at[b]).wait()

        def start_write(j, b):
            pltpu.async_copy(
                bufs.at[b], out_hbm.at[pl.ds(base + j * _CHUNK, _CHUNK)],
                wsem.at[b])

        def wait_write(b):
            pltpu.make_async_copy(
                bufs.at[b], out_hbm.at[pl.ds(0, _CHUNK)], wsem.at[b]).wait()

        for b in range(_AHEAD):
            start_gather(b, b)

        def body(m, carry):
            j0 = _NBUF * m
            for u in range(_NBUF):
                j = j0 + u
                pb = (u + _AHEAD) % _NBUF
                wait_gather(u)
                if u < _NBUF - _AHEAD:
                    @pl.when(m > 0)
                    def _():
                        wait_write(pb)
                        start_gather(j + _AHEAD, pb)

                    @pl.when(m == 0)
                    def _():
                        start_gather(j + _AHEAD, pb)
                else:
                    wait_write(pb)

                    @pl.when(m < nm - 1)
                    def _():
                        start_gather(j + _AHEAD, pb)
                start_write(j, u)
            return carry

        lax.fori_loop(0, nm, body, 0)
        for j in range(k - (_NBUF - _AHEAD), k):
            wait_write(j % _NBUF)

    return emb(idx_grouped, W)


def kernel(input_ids, W):
    B, L = input_ids.shape
    V, D = W.shape
    n = B * L
    idx = input_ids.reshape(_NW, n // (_NW * _CHUNK), _CHUNK).astype(jnp.int32)
    out = _embed_lookup(idx, W, n, D)
    return out.reshape(B, L, D)


# speedup vs baseline: 1.0226x; 1.0016x over previous
import functools

import jax
import jax.numpy as jnp
from jax import lax
from jax.experimental import pallas as pl
from jax.experimental.pallas import tpu as pltpu
from jax.experimental.pallas import tpu_sc as plsc

_INFO = plsc.get_sparse_core_info()
_NC = _INFO.num_cores
_NS = _INFO.num_subcores
_NW = _NC * _NS
_CHUNK = 80
_NBUF = 8
_AHEAD = 6


def _embed_lookup(idx_grouped, W, n_rows, d):
    k = idx_grouped.shape[1]
    per_w = k * _CHUNK
    assert k % _NBUF == 0
    nm = k // _NBUF

    @functools.partial(
        pl.kernel,
        mesh=plsc.VectorSubcoreMesh(core_axis_name="c", subcore_axis_name="s"),
        out_type=jax.ShapeDtypeStruct((n_rows, d), jnp.float32),
        scratch_types=[
            pltpu.VMEM((k, _CHUNK), jnp.int32),
            pltpu.VMEM((_NBUF, _CHUNK, d), jnp.float32),
            pltpu.SemaphoreType.DMA((_NBUF,)),
            pltpu.SemaphoreType.DMA((_NBUF,)),
        ],
    )
    def emb(idx_hbm, table_hbm, out_hbm, idx_v, bufs, gsem, wsem):
        wid = lax.axis_index("s") * _NC + lax.axis_index("c")
        base = wid * per_w
        pltpu.sync_copy(idx_hbm.at[wid], idx_v)

        def start_gather(j, b):
            pltpu.async_copy(table_hbm.at[idx_v.at[j]], bufs.at[b], gsem.at[b])

        def wait_gather(b):
            pltpu.make_async_copy(
                table_hbm.at[pl.ds(0, _CHUNK)], bufs.at[b], gsem.at[b]).wait()

        def start_write(j, b):
            pltpu.async_copy(
                bufs.at[b], out_hbm.at[pl.ds(base + j * _CHUNK, _CHUNK)],
                wsem.at[b])

        def wait_write(b):
            pltpu.make_async_copy(
                bufs.at[b], out_hbm.at[pl.ds(0, _CHUNK)], wsem.at[b]).wait()

        for b in range(_AHEAD):
            start_gather(b, b)

        def body(m, carry):
            j0 = _NBUF * m
            for u in range(_NBUF):
                j = j0 + u
                pb = (u + _AHEAD) % _NBUF
                wait_gather(u)
                if u < _NBUF - _AHEAD:
                    @pl.when(m > 0)
                    def _():
                        wait_write(pb)
                        start_gather(j + _AHEAD, pb)

                    @pl.when(m == 0)
                    def _():
                        start_gather(j + _AHEAD, pb)
                else:
                    wait_write(pb)

                    @pl.when(m < nm - 1)
                    def _():
                        start_gather(j + _AHEAD, pb)
                start_write(j, u)
            return carry

        lax.fori_loop(0, nm, body, 0)
        for j in range(k - (_NBUF - _AHEAD), k):
            wait_write(j % _NBUF)

    return emb(idx_grouped, W)


def kernel(input_ids, W):
    B, L = input_ids.shape
    V, D = W.shape
    n = B * L
    idx = input_ids.reshape(_NW, n // (_NW * _CHUNK), _CHUNK).astype(jnp.int32)
    out = _embed_lookup(idx, W, n, D)
    return out.reshape(B, L, D)
